# baseline (speedup 1.0000x reference)
import jax
import jax.numpy as jnp
from jax import lax
from jax.experimental import pallas as pl
from jax.experimental.pallas import tpu as pltpu
from jax.experimental.pallas import tpu_sc as plsc

_STATE_DIM = 64
_NUM_BINS = 256
_EMB_DIM = 128
_BATCH = 4096

_NC = 2
_NS = 16
_DIMS_C = _STATE_DIM // _NC
_ROWS_S = _BATCH // _NS
_HALF = _DIMS_C * _NUM_BINS
_STAGE = _HALF // _NS
_PHASES = 2
_RPP = _ROWS_S // _PHASES
_BPC = 1
_CHUNK = _BPC * _DIMS_C
_NCHUNK = _RPP // _BPC
_NBUF = 8
_K = 4
_WARM = 12
_LANES = 16
_VPC = _CHUNK // _LANES
_GRP = _DIMS_C // _LANES

assert _NCHUNK % _NBUF == 0 and 0 < _K < _NBUF <= _NCHUNK
assert _WARM >= _NBUF and (_NCHUNK - _K - _WARM) % _NBUF == 0


def _body(state_hbm, emb_hbm, smin_hbm, smax_hbm, out_hbm,
          table_sh, state_v, idx_v, min_v, den_v, rows_v, *sems):
  gsem = sems[:_NBUF]
  ssem = sems[_NBUF:2 * _NBUF]
  tsem = sems[2 * _NBUF]
  c = lax.axis_index("c")
  s = lax.axis_index("s")

  table_cp = pltpu.make_async_copy(
      emb_hbm.at[pl.ds(c * _HALF + s * _STAGE, _STAGE)],
      table_sh.at[pl.ds(s * _STAGE, _STAGE)], tsem)
  table_cp.start()
  pltpu.sync_copy(smin_hbm, min_v)
  pltpu.sync_copy(smax_hbm, den_v)
  for j in range(_STATE_DIM // _LANES):
    sl = pl.ds(j * _LANES, _LANES)
    den_v[sl] = den_v[sl] - min_v[sl] + 1e-08
  pltpu.sync_copy(state_hbm.at[pl.ds(s * _ROWS_S, _RPP)], state_v)

  def run_phase(row0, next_row0, warm):

    def compute_row(r, b, glob):
      for q in range(_VPC):
        j = q % _GRP
        sl = pl.ds(c * _DIMS_C + j * _LANES, _LANES)
        st = state_v[r * _BPC + q // _GRP, sl]
        norm = (st - min_v[sl]) / den_v[sl]
        norm = jnp.clip(norm, 0.0, 1.0)
        bins = (norm * float(_NUM_BINS - 1)).astype(jnp.int32)
        dimoff = (lax.iota(jnp.int32, _LANES) + (j * _LANES)) * _NUM_BINS
        idx = bins + dimoff
        if glob:
          idx = idx + c * _HALF
        idx_v[b, pl.ds(q * _LANES, _LANES)] = idx

    def g_copy(b, hbm):
      src = emb_hbm if hbm else table_sh
      return pltpu.make_async_copy(src.at[idx_v.at[b]], rows_v.at[b],
                                   gsem[b])

    def s_copies(r, b):
      return [
          pltpu.make_async_copy(
              rows_v.at[b, pl.ds(i * _DIMS_C, _DIMS_C)],
              out_hbm.at[s * _ROWS_S + row0 + r * _BPC + i,
                         pl.ds(c * _DIMS_C, _DIMS_C)],
              ssem[b])
          for i in range(_BPC)
      ]

    def step(r, b, bp, wait_s, g_hbm_wait=False, g_hbm_fire=False):
      if bp is not None:
        compute_row(r + _K, bp, glob=g_hbm_fire)
        if wait_s:
          for cp in s_copies(r + _K - _NBUF, bp):
            cp.wait()
        g_copy(bp, g_hbm_fire).start()
      g_copy(b, g_hbm_wait).wait()
      for cp in s_copies(r, b):
        cp.start()

    peel = warm if warm else _NBUF - _K

    for r in range(_K):
      compute_row(r, r % _NBUF, glob=r < warm)
      g_copy(r % _NBUF, r < warm).start()

    for r in range(peel):
      if warm and r == warm - _K:
        table_cp.wait()
        plsc.subcore_barrier()
      step(r, r % _NBUF, (r + _K) % _NBUF, wait_s=r >= _NBUF - _K,
           g_hbm_wait=r < warm, g_hbm_fire=r + _K < warm)

    nblocks = (_NCHUNK - _K - peel) // _NBUF

    def outer(m, carry):
      r_base = peel + m * _NBUF
      for i in range(_NBUF):
        b = (peel + i) % _NBUF
        step(r_base + i, b, (b + _K) % _NBUF, wait_s=True)
      return carry

    lax.fori_loop(0, nblocks, outer, 0)

    for r in range(_NCHUNK - _K, _NCHUNK):
      step(r, r % _NBUF, None, wait_s=False)
    if next_row0 is not None:
      pltpu.sync_copy(state_hbm.at[pl.ds(s * _ROWS_S + next_row0, _RPP)],
                      state_v)
    for r in range(_NCHUNK - _NBUF, _NCHUNK):
      for cp in s_copies(r, r % _NBUF):
        cp.wait()

  for p in range(_PHASES):
    run_phase(p * _RPP, (p + 1) * _RPP if p + 1 < _PHASES else None,
              _WARM if p == 0 else 0)


_encode = pl.kernel(
    _body,
    out_type=jax.ShapeDtypeStruct((_BATCH, _STATE_DIM, _EMB_DIM),
                                  jnp.float32),
    mesh=plsc.VectorSubcoreMesh(
        core_axis_name="c", subcore_axis_name="s",
        num_cores=_NC, num_subcores=_NS),
    scratch_types=[
        pltpu.VMEM_SHARED((_HALF, _EMB_DIM), jnp.float32),
        pltpu.VMEM((_RPP, _STATE_DIM), jnp.float32),
        pltpu.VMEM((_NBUF, _CHUNK), jnp.int32),
        pltpu.VMEM((_STATE_DIM,), jnp.float32),
        pltpu.VMEM((_STATE_DIM,), jnp.float32),
        pltpu.VMEM((_NBUF, _CHUNK, _EMB_DIM), jnp.float32),
    ] + [pltpu.SemaphoreType.DMA] * (2 * _NBUF + 1),
)


@jax.jit
def kernel(state, embedding, state_min, state_max):
  return _encode(state, embedding, state_min, state_max)


# speedup vs baseline: 1.0847x; 1.0847x over previous
import jax
import jax.numpy as jnp
from jax import lax
from jax.experimental import pallas as pl
from jax.experimental.pallas import tpu as pltpu
from jax.experimental.pallas import tpu_sc as plsc

_STATE_DIM = 64
_NUM_BINS = 256
_EMB_DIM = 128
_BATCH = 4096

_NC = 2
_NS = 16
_DIMS_C = _STATE_DIM // _NC
_ROWS_S = _BATCH // _NS
_HALF = _DIMS_C * _NUM_BINS
_STAGE = _HALF // _NS
_PHASES = 2
_RPP = _ROWS_S // _PHASES
_BPC = 1
_CHUNK = _BPC * _DIMS_C
_NCHUNK = _RPP // _BPC
_NBUF = 8
_K = 5
_LANES = 16
_VPC = _CHUNK // _LANES
_GRP = _DIMS_C // _LANES

assert _NCHUNK % _NBUF == 0 and 0 < _K < _NBUF <= _NCHUNK


def _body(state_hbm, emb_hbm, smin_hbm, smax_hbm, out_hbm,
          table_sh, state_v, idx_v, min_v, den_v, rows_v, *sems):
  gsem = sems[:_NBUF]
  ssem = sems[_NBUF:2 * _NBUF]
  tsem = sems[2 * _NBUF]
  c = lax.axis_index("c")
  s = lax.axis_index("s")

  table_cp = pltpu.make_async_copy(
      emb_hbm.at[pl.ds(c * _HALF + s * _STAGE, _STAGE)],
      table_sh.at[pl.ds(s * _STAGE, _STAGE)], tsem)
  table_cp.start()
  pltpu.sync_copy(smin_hbm, min_v)
  pltpu.sync_copy(smax_hbm, den_v)
  for j in range(_STATE_DIM // _LANES):
    sl = pl.ds(j * _LANES, _LANES)
    den_v[sl] = den_v[sl] - min_v[sl] + 1e-08
  pltpu.sync_copy(state_hbm.at[pl.ds(s * _ROWS_S, _RPP)], state_v)
  table_cp.wait()
  plsc.subcore_barrier()

  def run_phase(row0, next_row0):

    def compute_row(r, b):
      for q in range(_VPC):
        j = q % _GRP
        sl = pl.ds(c * _DIMS_C + j * _LANES, _LANES)
        st = state_v[r * _BPC + q // _GRP, sl]
        norm = (st - min_v[sl]) / den_v[sl]
        norm = jnp.clip(norm, 0.0, 1.0)
        bins = (norm * float(_NUM_BINS - 1)).astype(jnp.int32)
        dimoff = (lax.iota(jnp.int32, _LANES) + (j * _LANES)) * _NUM_BINS
        idx_v[b, pl.ds(q * _LANES, _LANES)] = bins + dimoff

    def g_copy(b):
      return pltpu.make_async_copy(table_sh.at[idx_v.at[b]], rows_v.at[b],
                                   gsem[b])

    def s_copies(r, b):
      return [
          pltpu.make_async_copy(
              rows_v.at[b, pl.ds(i * _DIMS_C, _DIMS_C)],
              out_hbm.at[s * _ROWS_S + row0 + r * _BPC + i,
                         pl.ds(c * _DIMS_C, _DIMS_C)],
              ssem[b])
          for i in range(_BPC)
      ]

    def step(r, b, bp, wait_s):
      if bp is not None:
        compute_row(r + _K, bp)
        if wait_s:
          for cp in s_copies(r + _K - _NBUF, bp):
            cp.wait()
        g_copy(bp).start()
      g_copy(b).wait()
      for cp in s_copies(r, b):
        cp.start()

    for r in range(_K):
      compute_row(r, r % _NBUF)
      g_copy(r % _NBUF).start()

    for r in range(_NBUF - _K):
      step(r, r % _NBUF, (r + _K) % _NBUF, wait_s=False)

    c0 = _NBUF - _K
    nblocks = (_NCHUNK - _NBUF) // _NBUF

    def outer(m, carry):
      r_base = c0 + m * _NBUF
      for i in range(_NBUF):
        b = (c0 + i) % _NBUF
        step(r_base + i, b, (b + _K) % _NBUF, wait_s=True)
      return carry

    lax.fori_loop(0, nblocks, outer, 0)

    for r in range(_NCHUNK - _K, _NCHUNK):
      step(r, r % _NBUF, None, wait_s=False)
    if next_row0 is not None:
      pltpu.sync_copy(state_hbm.at[pl.ds(s * _ROWS_S + next_row0, _RPP)],
                      state_v)
    for r in range(_NCHUNK - _NBUF, _NCHUNK):
      for cp in s_copies(r, r % _NBUF):
        cp.wait()

  for p in range(_PHASES):
    run_phase(p * _RPP, (p + 1) * _RPP if p + 1 < _PHASES else None)


_encode = pl.kernel(
    _body,
    out_type=jax.ShapeDtypeStruct((_BATCH, _STATE_DIM, _EMB_DIM),
                                  jnp.float32),
    mesh=plsc.VectorSubcoreMesh(
        core_axis_name="c", subcore_axis_name="s",
        num_cores=_NC, num_subcores=_NS),
    scratch_types=[
        pltpu.VMEM_SHARED((_HALF, _EMB_DIM), jnp.float32),
        pltpu.VMEM((_RPP, _STATE_DIM), jnp.float32),
        pltpu.VMEM((_NBUF, _CHUNK), jnp.int32),
        pltpu.VMEM((_STATE_DIM,), jnp.float32),
        pltpu.VMEM((_STATE_DIM,), jnp.float32),
        pltpu.VMEM((_NBUF, _CHUNK, _EMB_DIM), jnp.float32),
    ] + [pltpu.SemaphoreType.DMA] * (2 * _NBUF + 1),
)


@jax.jit
def kernel(state, embedding, state_min, state_max):
  return _encode(state, embedding, state_min, state_max)
